# Initial kernel scaffold; baseline (speedup 1.0000x reference)
import jax
import jax.numpy as jnp
from jax.experimental import pallas as pl


def kernel(x, u_idx, v_idx):
    raise NotImplementedError("write your pallas kernel here")


# speedup vs baseline: 2.7724x; 2.7724x over previous
import functools

import jax
import jax.numpy as jnp
from jax import lax
from jax.experimental import pallas as pl
from jax.experimental.pallas import tpu as pltpu
from jax.experimental.pallas import tpu_sc as plsc

_A = 5.0
_SIGMA = 1.0
_NUM_NEG = 32
_LN2 = 0.6931471805599453

_NC = 2
_NS = 16
_NW = _NC * _NS
_L = 16

_G = 80
_NG = _G // _L
_PITCH = 17


def _softlog(w):
    bits = plsc.bitcast(w, jnp.int32)
    e = (bits >> 23) & 0xFF
    m = plsc.bitcast((bits & 0x007FFFFF) | 0x3F800000, jnp.float32)
    big = m > 1.4142135
    m = jnp.where(big, 0.5 * m, m)
    e = jnp.where(big, e - 126, e - 127)
    t = (m - 1.0) / (m + 1.0)
    t2 = t * t
    p = t2 * jnp.float32(1.0 / 9.0) + jnp.float32(1.0 / 7.0)
    p = p * t2 + jnp.float32(0.2)
    p = p * t2 + jnp.float32(1.0 / 3.0)
    p = p * t2 + jnp.float32(1.0)
    return e.astype(jnp.float32) * jnp.float32(_LN2) + 2.0 * t * p


def _softsqrt(t):
    i = plsc.bitcast(t, jnp.int32)
    r = plsc.bitcast(jnp.int32(0x5F3759DF) - (i >> 1), jnp.float32)
    ht = 0.5 * t
    r = r * (1.5 - ht * r * r)
    r = r * (1.5 - ht * r * r)
    r = r * (1.5 - ht * r * r)
    return t * r


def _build_sc(n, d, e):
    assert d == 128 and e % _NW == 0
    ew = e // _NW
    assert ew % _G == 0
    nchunks = ew // _G
    nj = d // _L
    zrows = 16
    rows_per_sub = (n // (zrows * _NS)) * zrows
    tail_base = _NS * rows_per_sub
    tail_rows = n - tail_base
    assert rows_per_sub % zrows == 0 and tail_rows == zrows
    nzcopies = rows_per_sub // zrows

    def body(x_hbm, u3, v3, gout, eout,
             ubuf, vbuf, xu, xv, gu, gv, pbuf, fbuf, ebuf, zbuf, grad_sp,
             sem_u, sem_v):
        c = lax.axis_index("c")
        s = lax.axis_index("s")
        wid = s * _NC + c
        iota = lax.iota(jnp.int32, _L)
        jvec = jnp.where(iota == 0, jnp.float32(-1.0), jnp.float32(1.0))

        zero16 = jnp.zeros((_L,), jnp.float32)
        for r in range(zrows):
            for j in range(nj):
                zbuf[r, pl.ds(j * _L, _L)] = zero16

        def zero_body(i, carry):
            pltpu.sync_copy(
                zbuf, grad_sp.at[pl.ds(s * rows_per_sub + i * zrows, zrows)])
            return carry

        lax.fori_loop(0, nzcopies, zero_body, 0)

        @pl.when(s == _NS - 1)
        def _zero_tail():
            pltpu.sync_copy(zbuf, grad_sp.at[pl.ds(tail_base, zrows)])

        plsc.subcore_barrier()

        def chunk(ci, eacc):
            pltpu.sync_copy(u3.at[wid, ci], ubuf)
            pltpu.sync_copy(v3.at[wid, ci], vbuf)
            cu = pltpu.async_copy(x_hbm.at[ubuf], xu, sem_u)
            cv = pltpu.async_copy(x_hbm.at[vbuf], xv, sem_v)
            cu.wait()
            cv.wait()

            def p1(ee, carry):
                p = xu[ee, pl.ds(0, _L)] * xv[ee, pl.ds(0, _L)] * jvec
                for j in range(1, nj):
                    p = p + xu[ee, pl.ds(j * _L, _L)] * xv[ee, pl.ds(j * _L, _L)]
                plsc.store_scatter(
                    pbuf, [lax.broadcast(ee * _PITCH, (_L,)) + iota], p)
                return carry

            lax.fori_loop(0, _G, p1, 0)

            for g in range(_NG):
                base17 = (iota + g * _L) * _PITCH
                inner = plsc.load_gather(pbuf, [base17])
                for l in range(1, _L):
                    inner = inner + plsc.load_gather(pbuf, [base17 + l])
                inner = jnp.minimum(inner, jnp.float32(-1.0 - 1e-7))
                z = -inner
                sden = _softsqrt(z * z - 1.0)
                w = z + sden
                dist = _softlog(w)
                tanhv = 1.0 - 1.0 / (z * w)
                factor = -(jnp.float32(_A) * tanhv) / (sden + 1e-9)
                logz = _softlog(z)
                eacc = eacc + jnp.where(dist > 10.0,
                                        dist - jnp.float32(0.69314718), logz)
                fbuf[pl.ds(g * _L, _L)] = factor

            def p2(ee, carry):
                fv = plsc.load_gather(fbuf, [lax.broadcast(ee, (_L,))])
                fj = fv * jvec
                gu[ee, pl.ds(0, _L)] = fj * xv[ee, pl.ds(0, _L)]
                gv[ee, pl.ds(0, _L)] = fj * xu[ee, pl.ds(0, _L)]
                for j in range(1, nj):
                    gu[ee, pl.ds(j * _L, _L)] = fv * xv[ee, pl.ds(j * _L, _L)]
                    gv[ee, pl.ds(j * _L, _L)] = fv * xu[ee, pl.ds(j * _L, _L)]
                return carry

            lax.fori_loop(0, _G, p2, 0)

            pltpu.sync_copy(gu, grad_sp.at[ubuf], add=True)
            pltpu.sync_copy(gv, grad_sp.at[vbuf], add=True)
            return eacc

        eacc = lax.fori_loop(0, nchunks, chunk, jnp.zeros((_L,), jnp.float32))
        ebuf[...] = eacc
        plsc.subcore_barrier()

        pltpu.sync_copy(
            grad_sp.at[pl.ds(s * rows_per_sub, rows_per_sub)],
            gout.at[pl.ds(c * n + s * rows_per_sub, rows_per_sub)])

        @pl.when(s == _NS - 1)
        def _copy_tail():
            pltpu.sync_copy(
                grad_sp.at[pl.ds(tail_base, tail_rows)],
                gout.at[pl.ds(c * n + tail_base, tail_rows)])

        pltpu.sync_copy(ebuf, eout.at[wid])

    return pl.kernel(
        body,
        out_type=(jax.ShapeDtypeStruct((_NC * n, d), jnp.float32),
                  jax.ShapeDtypeStruct((_NW, _L), jnp.float32)),
        mesh=plsc.VectorSubcoreMesh(core_axis_name="c", subcore_axis_name="s",
                                    num_cores=_NC, num_subcores=_NS),
        compiler_params=pltpu.CompilerParams(needs_layout_passes=False),
        scratch_types=[
            pltpu.VMEM((_G,), jnp.int32),
            pltpu.VMEM((_G,), jnp.int32),
            pltpu.VMEM((_G, 128), jnp.float32),
            pltpu.VMEM((_G, 128), jnp.float32),
            pltpu.VMEM((_G, 128), jnp.float32),
            pltpu.VMEM((_G, 128), jnp.float32),
            pltpu.VMEM((_G * _PITCH,), jnp.float32),
            pltpu.VMEM((_G,), jnp.float32),
            pltpu.VMEM((_L,), jnp.float32),
            pltpu.VMEM((16, 128), jnp.float32),
            pltpu.VMEM_SHARED((n, 128), jnp.float32),
            pltpu.SemaphoreType.DMA,
            pltpu.SemaphoreType.DMA,
        ],
    )


def _combine_body(gp_ref, ep_ref, go_ref, eo_ref):
    go_ref[...] = gp_ref[0] + gp_ref[1]

    @pl.when(pl.program_id(0) == 0)
    def _():
        eo_ref[...] = (jnp.sum(ep_ref[...]) * jnp.float32(
            _A * _SIGMA / _NUM_NEG)).reshape(1, 1)


@functools.lru_cache(maxsize=None)
def _build(n, d, e):
    sc = _build_sc(n, d, e)
    blk = n // 10
    assert blk % 8 == 0
    nch = e // _NW // _G

    combine = pl.pallas_call(
        _combine_body,
        grid=(10,),
        in_specs=[
            pl.BlockSpec((2, blk, d), lambda i: (0, i, 0)),
            pl.BlockSpec((_NW, _L), lambda i: (0, 0)),
        ],
        out_specs=[
            pl.BlockSpec((blk, d), lambda i: (i, 0)),
            pl.BlockSpec((1, 1), lambda i: (0, 0)),
        ],
        out_shape=[
            jax.ShapeDtypeStruct((n, d), jnp.float32),
            jax.ShapeDtypeStruct((1, 1), jnp.float32),
        ],
    )

    def run(x, u_idx, v_idx):
        u3 = u_idx.reshape(_NW, nch, _G)
        v3 = v_idx.reshape(_NW, nch, _G)
        gparts, eparts = sc(x, u3, v3)
        grad, energy = combine(gparts.reshape(2, n, d), eparts)
        return jnp.reshape(energy, ()), grad

    return run


def kernel(x, u_idx, v_idx):
    n, d = x.shape
    e = u_idx.shape[0]
    energy, grad = _build(n, d, e)(x, u_idx, v_idx)
    return (energy, grad)


# speedup vs baseline: 4.3089x; 1.5542x over previous
import functools

import jax
import jax.numpy as jnp
from jax import lax
from jax.experimental import pallas as pl
from jax.experimental.pallas import tpu as pltpu
from jax.experimental.pallas import tpu_sc as plsc

_A = 5.0
_SIGMA = 1.0
_NUM_NEG = 32
_LN2 = 0.6931471805599453

_NC = 2
_NS = 16
_NW = _NC * _NS
_L = 16

_G = 80
_NG = _G // _L
_PITCH = 17


def _softlog(w):
    bits = plsc.bitcast(w, jnp.int32)
    e = (bits >> 23) & 0xFF
    m = plsc.bitcast((bits & 0x007FFFFF) | 0x3F800000, jnp.float32)
    big = m > 1.4142135
    m = jnp.where(big, 0.5 * m, m)
    e = jnp.where(big, e - 126, e - 127)
    t = (m - 1.0) / (m + 1.0)
    t2 = t * t
    p = t2 * jnp.float32(1.0 / 9.0) + jnp.float32(1.0 / 7.0)
    p = p * t2 + jnp.float32(0.2)
    p = p * t2 + jnp.float32(1.0 / 3.0)
    p = p * t2 + jnp.float32(1.0)
    return e.astype(jnp.float32) * jnp.float32(_LN2) + 2.0 * t * p


def _softsqrt(t):
    i = plsc.bitcast(t, jnp.int32)
    r = plsc.bitcast(jnp.int32(0x5F3759DF) - (i >> 1), jnp.float32)
    ht = 0.5 * t
    r = r * (1.5 - ht * r * r)
    r = r * (1.5 - ht * r * r)
    r = r * (1.5 - ht * r * r)
    return t * r


def _build_sc(n, d, e):
    assert d == 128 and e % _NW == 0
    ew = e // _NW
    assert ew % _G == 0
    nchunks = ew // _G
    nj = d // _L
    zrows = 16
    rows_per_sub = (n // (zrows * _NS)) * zrows
    tail_base = _NS * rows_per_sub
    tail_rows = n - tail_base
    assert rows_per_sub % zrows == 0 and tail_rows == zrows
    nzcopies = rows_per_sub // zrows

    def body(x_hbm, u3, v3, gout, eout,
             ub0, vb0, ub1, vb1, xu0, xv0, xu1, xv1, pbuf, fbuf, ebuf, zbuf,
             grad_sp, sg0, sg1, ss0, ss1):
        c = lax.axis_index("c")
        s = lax.axis_index("s")
        wid = s * _NC + c
        iota = lax.iota(jnp.int32, _L)
        jvec = jnp.where(iota == 0, jnp.float32(-1.0), jnp.float32(1.0))

        zero16 = jnp.zeros((_L,), jnp.float32)
        for r in range(zrows):
            for j in range(nj):
                zbuf[r, pl.ds(j * _L, _L)] = zero16

        def zero_body(i, carry):
            pltpu.sync_copy(
                zbuf, grad_sp.at[pl.ds(s * rows_per_sub + i * zrows, zrows)])
            return carry

        lax.fori_loop(0, nzcopies, zero_body, 0)

        @pl.when(s == _NS - 1)
        def _zero_tail():
            pltpu.sync_copy(zbuf, grad_sp.at[pl.ds(tail_base, zrows)])

        plsc.subcore_barrier()

        xus = (xu0, xu1)
        xvs = (xv0, xv1)
        ubs = (ub0, ub1)
        vbs = (vb0, vb1)
        sgs = (sg0, sg1)
        sss = (ss0, ss1)

        def gather_issue(ci, p):
            pltpu.sync_copy(u3.at[wid, ci], ubs[p])
            pltpu.sync_copy(v3.at[wid, ci], vbs[p])
            pltpu.async_copy(x_hbm.at[ubs[p]], xus[p], sgs[p])
            pltpu.async_copy(x_hbm.at[vbs[p]], xvs[p], sgs[p])

        def gather_wait(p):
            pltpu.make_async_copy(x_hbm.at[ubs[p]], xus[p], sgs[p]).wait()
            pltpu.make_async_copy(x_hbm.at[vbs[p]], xvs[p], sgs[p]).wait()

        def scatter_issue(p):
            pltpu.async_copy(xvs[p], grad_sp.at[ubs[p]], sss[p], add=True)
            pltpu.async_copy(xus[p], grad_sp.at[vbs[p]], sss[p], add=True)

        def scatter_wait(p):
            pltpu.make_async_copy(xvs[p], grad_sp.at[ubs[p]], sss[p]).wait()
            pltpu.make_async_copy(xus[p], grad_sp.at[vbs[p]], sss[p]).wait()

        def compute(p, eacc):
            xu, xv = xus[p], xvs[p]

            def p1(ee, carry):
                pp = xu[ee, pl.ds(0, _L)] * xv[ee, pl.ds(0, _L)] * jvec
                for j in range(1, nj):
                    pp = pp + xu[ee, pl.ds(j * _L, _L)] * xv[ee, pl.ds(j * _L, _L)]
                plsc.store_scatter(
                    pbuf, [lax.broadcast(ee * _PITCH, (_L,)) + iota], pp)
                return carry

            lax.fori_loop(0, _G, p1, 0)

            for g in range(_NG):
                base17 = (iota + g * _L) * _PITCH
                inner = plsc.load_gather(pbuf, [base17])
                for l in range(1, _L):
                    inner = inner + plsc.load_gather(pbuf, [base17 + l])
                inner = jnp.minimum(inner, jnp.float32(-1.0 - 1e-7))
                z = -inner
                sden = _softsqrt(z * z - 1.0)
                w = z + sden
                dist = _softlog(w)
                tanhv = 1.0 - 1.0 / (z * w)
                factor = -(jnp.float32(_A) * tanhv) / (sden + 1e-9)
                logz = _softlog(z)
                eacc = eacc + jnp.where(dist > 10.0,
                                        dist - jnp.float32(0.69314718), logz)
                fbuf[pl.ds(g * _L, _L)] = factor

            def p2(ee, carry):
                fv = plsc.load_gather(fbuf, [lax.broadcast(ee, (_L,))])
                fj = fv * jvec
                for j in range(nj):
                    ff = fj if j == 0 else fv
                    a = xu[ee, pl.ds(j * _L, _L)]
                    b = xv[ee, pl.ds(j * _L, _L)]
                    xv[ee, pl.ds(j * _L, _L)] = ff * b
                    xu[ee, pl.ds(j * _L, _L)] = ff * a
                return carry

            lax.fori_loop(0, _G, p2, 0)
            return eacc

        def step(p, ci, eacc):
            q = 1 - p

            @pl.when((ci >= 1) & (ci <= nchunks - 2))
            def _wait_sc():
                scatter_wait(q)

            @pl.when(ci <= nchunks - 2)
            def _issue_g():
                gather_issue(ci + 1, q)

            gather_wait(p)
            eacc = compute(p, eacc)
            scatter_issue(p)
            return eacc

        gather_issue(0, 0)
        eacc = lax.fori_loop(
            0, nchunks,
            lambda ci, ea: lax.cond(ci % 2 == 0,
                                    lambda e2: step(0, ci, e2),
                                    lambda e2: step(1, ci, e2),
                                    ea),
            jnp.zeros((_L,), jnp.float32))
        scatter_wait(0)
        scatter_wait(1)
        ebuf[...] = eacc
        plsc.subcore_barrier()

        pltpu.sync_copy(
            grad_sp.at[pl.ds(s * rows_per_sub, rows_per_sub)],
            gout.at[pl.ds(c * n + s * rows_per_sub, rows_per_sub)])

        @pl.when(s == _NS - 1)
        def _copy_tail():
            pltpu.sync_copy(
                grad_sp.at[pl.ds(tail_base, tail_rows)],
                gout.at[pl.ds(c * n + tail_base, tail_rows)])

        pltpu.sync_copy(ebuf, eout.at[wid])

    return pl.kernel(
        body,
        out_type=(jax.ShapeDtypeStruct((_NC * n, d), jnp.float32),
                  jax.ShapeDtypeStruct((_NW, _L), jnp.float32)),
        mesh=plsc.VectorSubcoreMesh(core_axis_name="c", subcore_axis_name="s",
                                    num_cores=_NC, num_subcores=_NS),
        compiler_params=pltpu.CompilerParams(needs_layout_passes=False),
        scratch_types=[
            pltpu.VMEM((_G,), jnp.int32),
            pltpu.VMEM((_G,), jnp.int32),
            pltpu.VMEM((_G,), jnp.int32),
            pltpu.VMEM((_G,), jnp.int32),
            pltpu.VMEM((_G, 128), jnp.float32),
            pltpu.VMEM((_G, 128), jnp.float32),
            pltpu.VMEM((_G, 128), jnp.float32),
            pltpu.VMEM((_G, 128), jnp.float32),
            pltpu.VMEM((_G * _PITCH,), jnp.float32),
            pltpu.VMEM((_G,), jnp.float32),
            pltpu.VMEM((_L,), jnp.float32),
            pltpu.VMEM((16, 128), jnp.float32),
            pltpu.VMEM_SHARED((n, 128), jnp.float32),
            pltpu.SemaphoreType.DMA,
            pltpu.SemaphoreType.DMA,
            pltpu.SemaphoreType.DMA,
            pltpu.SemaphoreType.DMA,
        ],
    )


def _combine_body(gp_ref, ep_ref, go_ref, eo_ref):
    go_ref[...] = gp_ref[0] + gp_ref[1]

    @pl.when(pl.program_id(0) == 0)
    def _():
        eo_ref[...] = (jnp.sum(ep_ref[...]) * jnp.float32(
            _A * _SIGMA / _NUM_NEG)).reshape(1, 1)


@functools.lru_cache(maxsize=None)
def _build(n, d, e):
    sc = _build_sc(n, d, e)
    blk = n // 10
    assert blk % 8 == 0
    nch = e // _NW // _G

    combine = pl.pallas_call(
        _combine_body,
        grid=(10,),
        in_specs=[
            pl.BlockSpec((2, blk, d), lambda i: (0, i, 0)),
            pl.BlockSpec((_NW, _L), lambda i: (0, 0)),
        ],
        out_specs=[
            pl.BlockSpec((blk, d), lambda i: (i, 0)),
            pl.BlockSpec((1, 1), lambda i: (0, 0)),
        ],
        out_shape=[
            jax.ShapeDtypeStruct((n, d), jnp.float32),
            jax.ShapeDtypeStruct((1, 1), jnp.float32),
        ],
    )

    def run(x, u_idx, v_idx):
        u3 = u_idx.reshape(_NW, nch, _G)
        v3 = v_idx.reshape(_NW, nch, _G)
        gparts, eparts = sc(x, u3, v3)
        grad, energy = combine(gparts.reshape(2, n, d), eparts)
        return jnp.reshape(energy, ()), grad

    return run


def kernel(x, u_idx, v_idx):
    n, d = x.shape
    e = u_idx.shape[0]
    energy, grad = _build(n, d, e)(x, u_idx, v_idx)
    return (energy, grad)


# speedup vs baseline: 6.9046x; 1.6024x over previous
import functools

import jax
import jax.numpy as jnp
from jax import lax
from jax.experimental import pallas as pl
from jax.experimental.pallas import tpu as pltpu
from jax.experimental.pallas import tpu_sc as plsc

_A = 5.0
_SIGMA = 1.0
_NUM_NEG = 32
_LN2 = 0.6931471805599453

_NC = 2
_NS = 16
_NW = _NC * _NS
_L = 16

_G = 80
_NG = _G // _L
_PITCH = 17


def _softlog(w):
    bits = plsc.bitcast(w, jnp.int32)
    e = (bits >> 23) & 0xFF
    m = plsc.bitcast((bits & 0x007FFFFF) | 0x3F800000, jnp.float32)
    big = m > 1.4142135
    m = jnp.where(big, 0.5 * m, m)
    e = jnp.where(big, e - 126, e - 127)
    t = (m - 1.0) / (m + 1.0)
    t2 = t * t
    p = t2 * jnp.float32(1.0 / 9.0) + jnp.float32(1.0 / 7.0)
    p = p * t2 + jnp.float32(0.2)
    p = p * t2 + jnp.float32(1.0 / 3.0)
    p = p * t2 + jnp.float32(1.0)
    return e.astype(jnp.float32) * jnp.float32(_LN2) + 2.0 * t * p


def _softsqrt(t):
    i = plsc.bitcast(t, jnp.int32)
    r = plsc.bitcast(jnp.int32(0x5F3759DF) - (i >> 1), jnp.float32)
    ht = 0.5 * t
    r = r * (1.5 - ht * r * r)
    r = r * (1.5 - ht * r * r)
    r = r * (1.5 - ht * r * r)
    return t * r


def _build_sc(n, d, e):
    assert d == 128 and e % _NW == 0
    ew = e // _NW
    assert ew % _G == 0
    nchunks = ew // _G
    nj = d // _L
    zrows = 16
    rows_per_sub = (n // (zrows * _NS)) * zrows
    tail_base = _NS * rows_per_sub
    tail_rows = n - tail_base
    assert rows_per_sub % zrows == 0 and tail_rows == zrows
    nzcopies = rows_per_sub // zrows

    def body(x_hbm, w3, gout, eout,
             wb0, wb1, xb0, xb1, pbuf, fbuf, ebuf, zbuf,
             grad_sp, sg0, sg1, ss0, ss1):
        c = lax.axis_index("c")
        s = lax.axis_index("s")
        wid = s * _NC + c
        iota = lax.iota(jnp.int32, _L)
        jvec = jnp.where(iota == 0, jnp.float32(-1.0), jnp.float32(1.0))

        zero16 = jnp.zeros((_L,), jnp.float32)
        for r in range(zrows):
            for j in range(nj):
                zbuf[r, pl.ds(j * _L, _L)] = zero16

        def zero_body(i, carry):
            pltpu.sync_copy(
                zbuf, grad_sp.at[pl.ds(s * rows_per_sub + i * zrows, zrows)])
            return carry

        lax.fori_loop(0, nzcopies, zero_body, 0)

        @pl.when(s == _NS - 1)
        def _zero_tail():
            pltpu.sync_copy(zbuf, grad_sp.at[pl.ds(tail_base, zrows)])

        plsc.subcore_barrier()

        xbs = (xb0, xb1)
        wbs = (wb0, wb1)
        sgs = (sg0, sg1)
        sss = (ss0, ss1)

        def gather_issue(ci, p):
            pltpu.sync_copy(w3.at[wid, ci], wbs[p])
            pltpu.async_copy(x_hbm.at[wbs[p]], xbs[p], sgs[p])

        def gather_wait(p):
            pltpu.make_async_copy(x_hbm.at[wbs[p]], xbs[p], sgs[p]).wait()

        def scatter_issue(p):
            pltpu.async_copy(xbs[p], grad_sp.at[wbs[p]], sss[p], add=True)

        def scatter_wait(p):
            pltpu.make_async_copy(xbs[p], grad_sp.at[wbs[p]], sss[p]).wait()

        def compute(p, eacc):
            xb = xbs[p]

            def p1(ee, carry):
                pp = xb[2 * ee, pl.ds(0, _L)] * xb[2 * ee + 1, pl.ds(0, _L)] * jvec
                for j in range(1, nj):
                    pp = pp + (xb[2 * ee, pl.ds(j * _L, _L)]
                               * xb[2 * ee + 1, pl.ds(j * _L, _L)])
                plsc.store_scatter(
                    pbuf, [lax.broadcast(ee * _PITCH, (_L,)) + iota], pp)
                return carry

            lax.fori_loop(0, _G, p1, 0)

            for g in range(_NG):
                base17 = (iota + g * _L) * _PITCH
                inner = plsc.load_gather(pbuf, [base17])
                for l in range(1, _L):
                    inner = inner + plsc.load_gather(pbuf, [base17 + l])
                inner = jnp.minimum(inner, jnp.float32(-1.0 - 1e-7))
                z = -inner
                sden = _softsqrt(z * z - 1.0)
                w = z + sden
                dist = _softlog(w)
                tanhv = 1.0 - 1.0 / (z * w)
                factor = -(jnp.float32(_A) * tanhv) / (sden + 1e-9)
                logz = _softlog(z)
                eacc = eacc + jnp.where(dist > 10.0,
                                        dist - jnp.float32(0.69314718), logz)
                fbuf[pl.ds(g * _L, _L)] = factor

            def p2(ee, carry):
                fv = plsc.load_gather(fbuf, [lax.broadcast(ee, (_L,))])
                fj = fv * jvec
                for j in range(nj):
                    ff = fj if j == 0 else fv
                    a = xb[2 * ee, pl.ds(j * _L, _L)]
                    b = xb[2 * ee + 1, pl.ds(j * _L, _L)]
                    xb[2 * ee, pl.ds(j * _L, _L)] = ff * b
                    xb[2 * ee + 1, pl.ds(j * _L, _L)] = ff * a
                return carry

            lax.fori_loop(0, _G, p2, 0)
            return eacc

        def step(p, ci, eacc):
            q = 1 - p

            @pl.when((ci >= 1) & (ci <= nchunks - 2))
            def _wait_sc():
                scatter_wait(q)

            @pl.when(ci <= nchunks - 2)
            def _issue_g():
                gather_issue(ci + 1, q)

            gather_wait(p)
            eacc = compute(p, eacc)
            scatter_issue(p)
            return eacc

        gather_issue(0, 0)
        eacc = lax.fori_loop(
            0, nchunks,
            lambda ci, ea: lax.cond(ci % 2 == 0,
                                    lambda e2: step(0, ci, e2),
                                    lambda e2: step(1, ci, e2),
                                    ea),
            jnp.zeros((_L,), jnp.float32))
        scatter_wait(0)
        scatter_wait(1)
        ebuf[...] = eacc
        plsc.subcore_barrier()

        pltpu.sync_copy(
            grad_sp.at[pl.ds(s * rows_per_sub, rows_per_sub)],
            gout.at[pl.ds(c * n + s * rows_per_sub, rows_per_sub)])

        @pl.when(s == _NS - 1)
        def _copy_tail():
            pltpu.sync_copy(
                grad_sp.at[pl.ds(tail_base, tail_rows)],
                gout.at[pl.ds(c * n + tail_base, tail_rows)])

        pltpu.sync_copy(ebuf, eout.at[wid])

    return pl.kernel(
        body,
        out_type=(jax.ShapeDtypeStruct((_NC * n, d), jnp.float32),
                  jax.ShapeDtypeStruct((_NW, _L), jnp.float32)),
        mesh=plsc.VectorSubcoreMesh(core_axis_name="c", subcore_axis_name="s",
                                    num_cores=_NC, num_subcores=_NS),
        compiler_params=pltpu.CompilerParams(needs_layout_passes=False),
        scratch_types=[
            pltpu.VMEM((2 * _G,), jnp.int32),
            pltpu.VMEM((2 * _G,), jnp.int32),
            pltpu.VMEM((2 * _G, 128), jnp.float32),
            pltpu.VMEM((2 * _G, 128), jnp.float32),
            pltpu.VMEM((_G * _PITCH,), jnp.float32),
            pltpu.VMEM((_G,), jnp.float32),
            pltpu.VMEM((_L,), jnp.float32),
            pltpu.VMEM((16, 128), jnp.float32),
            pltpu.VMEM_SHARED((n, 128), jnp.float32),
            pltpu.SemaphoreType.DMA,
            pltpu.SemaphoreType.DMA,
            pltpu.SemaphoreType.DMA,
            pltpu.SemaphoreType.DMA,
        ],
    )


def _combine_body(gp_ref, ep_ref, go_ref, eo_ref):
    go_ref[...] = gp_ref[0] + gp_ref[1]

    @pl.when(pl.program_id(0) == 0)
    def _():
        eo_ref[...] = (jnp.sum(ep_ref[...]) * jnp.float32(
            _A * _SIGMA / _NUM_NEG)).reshape(1, 1)


@functools.lru_cache(maxsize=None)
def _build(n, d, e):
    sc = _build_sc(n, d, e)
    blk = n // 10
    assert blk % 8 == 0
    nch = e // _NW // _G

    combine = pl.pallas_call(
        _combine_body,
        grid=(10,),
        in_specs=[
            pl.BlockSpec((2, blk, d), lambda i: (0, i, 0)),
            pl.BlockSpec((_NW, _L), lambda i: (0, 0)),
        ],
        out_specs=[
            pl.BlockSpec((blk, d), lambda i: (i, 0)),
            pl.BlockSpec((1, 1), lambda i: (0, 0)),
        ],
        out_shape=[
            jax.ShapeDtypeStruct((n, d), jnp.float32),
            jax.ShapeDtypeStruct((1, 1), jnp.float32),
        ],
    )

    def run(x, u_idx, v_idx):
        u3 = u_idx.reshape(_NW, nch, _G)
        v3 = v_idx.reshape(_NW, nch, _G)
        w3 = jnp.stack([u3, v3], axis=-1).reshape(_NW, nch, 2 * _G)
        gparts, eparts = sc(x, w3)
        grad, energy = combine(gparts.reshape(2, n, d), eparts)
        return jnp.reshape(energy, ()), grad

    return run


def kernel(x, u_idx, v_idx):
    n, d = x.shape
    e = u_idx.shape[0]
    energy, grad = _build(n, d, e)(x, u_idx, v_idx)
    return (energy, grad)


# speedup vs baseline: 7.6356x; 1.1059x over previous
import functools

import jax
import jax.numpy as jnp
from jax import lax
from jax.experimental import pallas as pl
from jax.experimental.pallas import tpu as pltpu
from jax.experimental.pallas import tpu_sc as plsc

_A = 5.0
_SIGMA = 1.0
_NUM_NEG = 32
_LN2 = 0.6931471805599453

_NC = 2
_NS = 16
_NW = _NC * _NS
_L = 16

_G = 80
_NG = _G // _L
_PITCH = 17


def _softlog(w):
    bits = plsc.bitcast(w, jnp.int32)
    e = (bits >> 23) & 0xFF
    m = plsc.bitcast((bits & 0x007FFFFF) | 0x3F800000, jnp.float32)
    big = m > 1.4142135
    m = jnp.where(big, 0.5 * m, m)
    e = jnp.where(big, e - 126, e - 127)
    t = (m - 1.0) / (m + 1.0)
    t2 = t * t
    p = t2 * jnp.float32(1.0 / 9.0) + jnp.float32(1.0 / 7.0)
    p = p * t2 + jnp.float32(0.2)
    p = p * t2 + jnp.float32(1.0 / 3.0)
    p = p * t2 + jnp.float32(1.0)
    return e.astype(jnp.float32) * jnp.float32(_LN2) + 2.0 * t * p


def _softsqrt(t):
    i = plsc.bitcast(t, jnp.int32)
    r = plsc.bitcast(jnp.int32(0x5F3759DF) - (i >> 1), jnp.float32)
    ht = 0.5 * t
    r = r * (1.5 - ht * r * r)
    r = r * (1.5 - ht * r * r)
    r = r * (1.5 - ht * r * r)
    return t * r


def _build_sc(n, d, e):
    assert d == 128 and e % _NW == 0
    ew = e // _NW
    assert ew % _G == 0
    nchunks = ew // _G
    nj = d // _L
    zrows = 16
    rows_per_sub = (n // (zrows * _NS)) * zrows
    tail_base = _NS * rows_per_sub
    tail_rows = n - tail_base
    assert rows_per_sub % zrows == 0 and tail_rows == zrows
    nzcopies = rows_per_sub // zrows

    def body(x_hbm, w3, gout, eout,
             wb0, wb1, xb0, xb1, pbuf, fbuf, ebuf, zbuf,
             grad_sp, sg0, sg1, ss0, ss1):
        c = lax.axis_index("c")
        s = lax.axis_index("s")
        wid = s * _NC + c
        iota = lax.iota(jnp.int32, _L)
        jvec = jnp.where(iota == 0, jnp.float32(-1.0), jnp.float32(1.0))

        zero16 = jnp.zeros((_L,), jnp.float32)
        for r in range(zrows):
            for j in range(nj):
                zbuf[r, pl.ds(j * _L, _L)] = zero16

        def zero_body(i, carry):
            pltpu.sync_copy(
                zbuf, grad_sp.at[pl.ds(s * rows_per_sub + i * zrows, zrows)])
            return carry

        lax.fori_loop(0, nzcopies, zero_body, 0)

        @pl.when(s == _NS - 1)
        def _zero_tail():
            pltpu.sync_copy(zbuf, grad_sp.at[pl.ds(tail_base, zrows)])

        plsc.subcore_barrier()

        xbs = (xb0, xb1)
        wbs = (wb0, wb1)
        sgs = (sg0, sg1)
        sss = (ss0, ss1)

        def gather_issue(ci, p):
            pltpu.sync_copy(w3.at[wid, ci], wbs[p])
            pltpu.async_copy(x_hbm.at[wbs[p]], xbs[p], sgs[p])

        def gather_wait(p):
            pltpu.make_async_copy(x_hbm.at[wbs[p]], xbs[p], sgs[p]).wait()

        def scatter_issue(p):
            pltpu.async_copy(xbs[p], grad_sp.at[wbs[p]], sss[p], add=True)

        def scatter_wait(p):
            pltpu.make_async_copy(xbs[p], grad_sp.at[wbs[p]], sss[p]).wait()

        def compute(p, eacc):
            xb = xbs[p]

            @plsc.parallel_loop(0, _G, 1, unroll=4)
            def p1(ee):
                pp = xb[2 * ee, pl.ds(0, _L)] * xb[2 * ee + 1, pl.ds(0, _L)] * jvec
                for j in range(1, nj):
                    pp = pp + (xb[2 * ee, pl.ds(j * _L, _L)]
                               * xb[2 * ee + 1, pl.ds(j * _L, _L)])
                plsc.store_scatter(
                    pbuf, [lax.broadcast(ee * _PITCH, (_L,)) + iota], pp)

            for g in range(_NG):
                base17 = (iota + g * _L) * _PITCH
                inner = plsc.load_gather(pbuf, [base17])
                for l in range(1, _L):
                    inner = inner + plsc.load_gather(pbuf, [base17 + l])
                inner = jnp.minimum(inner, jnp.float32(-1.0 - 1e-7))
                z = -inner
                sden = _softsqrt(z * z - 1.0)
                w = z + sden
                dist = _softlog(w)
                tanhv = 1.0 - 1.0 / (z * w)
                factor = -(jnp.float32(_A) * tanhv) / (sden + 1e-9)
                logz = _softlog(z)
                eacc = eacc + jnp.where(dist > 10.0,
                                        dist - jnp.float32(0.69314718), logz)
                fbuf[pl.ds(g * _L, _L)] = factor

            @plsc.parallel_loop(0, _G, 1, unroll=4)
            def p2(ee):
                fv = plsc.load_gather(fbuf, [lax.broadcast(ee, (_L,))])
                fj = fv * jvec
                for j in range(nj):
                    ff = fj if j == 0 else fv
                    a = xb[2 * ee, pl.ds(j * _L, _L)]
                    b = xb[2 * ee + 1, pl.ds(j * _L, _L)]
                    xb[2 * ee, pl.ds(j * _L, _L)] = ff * b
                    xb[2 * ee + 1, pl.ds(j * _L, _L)] = ff * a

            return eacc

        def step(p, ci, eacc):
            q = 1 - p

            @pl.when((ci >= 1) & (ci <= nchunks - 2))
            def _wait_sc():
                scatter_wait(q)

            @pl.when(ci <= nchunks - 2)
            def _issue_g():
                gather_issue(ci + 1, q)

            gather_wait(p)
            eacc = compute(p, eacc)
            scatter_issue(p)
            return eacc

        gather_issue(0, 0)
        eacc = lax.fori_loop(
            0, nchunks,
            lambda ci, ea: lax.cond(ci % 2 == 0,
                                    lambda e2: step(0, ci, e2),
                                    lambda e2: step(1, ci, e2),
                                    ea),
            jnp.zeros((_L,), jnp.float32))
        scatter_wait(0)
        scatter_wait(1)
        ebuf[...] = eacc
        plsc.subcore_barrier()

        pltpu.sync_copy(
            grad_sp.at[pl.ds(s * rows_per_sub, rows_per_sub)],
            gout.at[pl.ds(c * n + s * rows_per_sub, rows_per_sub)])

        @pl.when(s == _NS - 1)
        def _copy_tail():
            pltpu.sync_copy(
                grad_sp.at[pl.ds(tail_base, tail_rows)],
                gout.at[pl.ds(c * n + tail_base, tail_rows)])

        pltpu.sync_copy(ebuf, eout.at[wid])

    return pl.kernel(
        body,
        out_type=(jax.ShapeDtypeStruct((_NC * n, d), jnp.float32),
                  jax.ShapeDtypeStruct((_NW, _L), jnp.float32)),
        mesh=plsc.VectorSubcoreMesh(core_axis_name="c", subcore_axis_name="s",
                                    num_cores=_NC, num_subcores=_NS),
        compiler_params=pltpu.CompilerParams(needs_layout_passes=False),
        scratch_types=[
            pltpu.VMEM((2 * _G,), jnp.int32),
            pltpu.VMEM((2 * _G,), jnp.int32),
            pltpu.VMEM((2 * _G, 128), jnp.float32),
            pltpu.VMEM((2 * _G, 128), jnp.float32),
            pltpu.VMEM((_G * _PITCH,), jnp.float32),
            pltpu.VMEM((_G,), jnp.float32),
            pltpu.VMEM((_L,), jnp.float32),
            pltpu.VMEM((16, 128), jnp.float32),
            pltpu.VMEM_SHARED((n, 128), jnp.float32),
            pltpu.SemaphoreType.DMA,
            pltpu.SemaphoreType.DMA,
            pltpu.SemaphoreType.DMA,
            pltpu.SemaphoreType.DMA,
        ],
    )


def _combine_body(gp_ref, ep_ref, go_ref, eo_ref):
    go_ref[...] = gp_ref[0] + gp_ref[1]

    @pl.when(pl.program_id(0) == 0)
    def _():
        eo_ref[...] = (jnp.sum(ep_ref[...]) * jnp.float32(
            _A * _SIGMA / _NUM_NEG)).reshape(1, 1)


@functools.lru_cache(maxsize=None)
def _build(n, d, e):
    sc = _build_sc(n, d, e)
    blk = n // 10
    assert blk % 8 == 0
    nch = e // _NW // _G

    combine = pl.pallas_call(
        _combine_body,
        grid=(10,),
        in_specs=[
            pl.BlockSpec((2, blk, d), lambda i: (0, i, 0)),
            pl.BlockSpec((_NW, _L), lambda i: (0, 0)),
        ],
        out_specs=[
            pl.BlockSpec((blk, d), lambda i: (i, 0)),
            pl.BlockSpec((1, 1), lambda i: (0, 0)),
        ],
        out_shape=[
            jax.ShapeDtypeStruct((n, d), jnp.float32),
            jax.ShapeDtypeStruct((1, 1), jnp.float32),
        ],
    )

    def run(x, u_idx, v_idx):
        u3 = u_idx.reshape(_NW, nch, _G)
        v3 = v_idx.reshape(_NW, nch, _G)
        w3 = jnp.stack([u3, v3], axis=-1).reshape(_NW, nch, 2 * _G)
        gparts, eparts = sc(x, w3)
        grad, energy = combine(gparts.reshape(2, n, d), eparts)
        return jnp.reshape(energy, ()), grad

    return run


def kernel(x, u_idx, v_idx):
    n, d = x.shape
    e = u_idx.shape[0]
    energy, grad = _build(n, d, e)(x, u_idx, v_idx)
    return (energy, grad)
